# baseline (speedup 1.0000x reference)
import functools

import jax
import jax.numpy as jnp
from jax import lax
from jax.experimental import pallas as pl
from jax.experimental.pallas import tpu as pltpu
from jax.experimental.pallas import tpu_sc as plsc

BATCH = 4096
HIST = 200
FEAT = 32

NUM_CORES = 2
NUM_SUBCORES = 16
NW = NUM_CORES * NUM_SUBCORES
B_PER_W = BATCH // NW
T_CHUNK = 8
NCHUNK = HIST // T_CHUNK
NBUF = 3


def _embed_kernel(idx_hbm, table_hbm, out_hbm, idx_tv, bufs, sem0, sem1, sem2):
    sems = (sem0, sem1, sem2)
    wid = lax.axis_index("s") * NUM_CORES + lax.axis_index("c")
    b0 = wid * B_PER_W
    pltpu.sync_copy(idx_hbm.at[:, pl.ds(b0, B_PER_W)], idx_tv)

    def start_gather(g):
        s = g % NBUF
        return [
            pltpu.async_copy(
                table_hbm.at[idx_tv.at[g * T_CHUNK + tt]],
                bufs.at[s].at[tt],
                sems[s],
            )
            for tt in range(T_CHUNK)
        ]

    def start_out(g):
        s = g % NBUF
        return [
            pltpu.async_copy(
                bufs.at[s].at[tt],
                out_hbm.at[pl.ds(b0, B_PER_W), g * T_CHUNK + tt],
                sems[s],
            )
            for tt in range(T_CHUNK)
        ]

    gh = {0: start_gather(0), 1: start_gather(1)}
    oh = {}
    for g in range(NCHUNK):
        if g + 2 < NCHUNK:
            if g >= 1:
                for h in oh[g - 1]:
                    h.wait()
            gh[g + 2] = start_gather(g + 2)
        for h in gh[g]:
            h.wait()
        oh[g] = start_out(g)
    for g in range(max(0, NCHUNK - 3), NCHUNK):
        for h in oh[g]:
            h.wait()


@jax.jit
def _embed(idx_t, table):
    k = functools.partial(
        pl.kernel,
        mesh=plsc.VectorSubcoreMesh(core_axis_name="c", subcore_axis_name="s"),
        out_type=jax.ShapeDtypeStruct((BATCH, HIST, FEAT), jnp.float32),
        scratch_types=[
            pltpu.VMEM((HIST, B_PER_W), jnp.int32),
            pltpu.VMEM((NBUF, T_CHUNK, B_PER_W, FEAT), jnp.float32),
            pltpu.SemaphoreType.DMA,
            pltpu.SemaphoreType.DMA,
            pltpu.SemaphoreType.DMA,
        ],
        compiler_params=pltpu.CompilerParams(use_tc_tiling_on_sc=False),
    )(_embed_kernel)
    return k(idx_t, table)


def kernel(inputs, embedding):
    return _embed(inputs.T.astype(jnp.int32), embedding)
